# baseline (speedup 1.0000x reference)
import functools

import jax
import jax.numpy as jnp
from jax import lax
from jax.experimental import pallas as pl
from jax.experimental.pallas import tpu as pltpu
from jax.experimental.pallas import tpu_sc as plsc

_N = 100000
_G = 1024
_Z = 89
_TBL = 128
_NC = 2
_NS = 16
_NW = _NC * _NS
_CW = 128
_ROWS_PER_W = 25
_NPAD = _NW * _ROWS_PER_W * _CW
_VECS_PER_ROW = _CW // 16


def _sc_body(nt_hbm, bt_hbm, tbl_hbm, out_hbm,
             nt_v, bt_v, val_v, tbl_v, zero_v, acc_sh,
             sem_nt, sem_bt, sem_sc):
    cid = lax.axis_index("c")
    sid = lax.axis_index("s")
    wid = sid * _NC + cid

    nt_dma = pltpu.async_copy(nt_hbm.at[wid], nt_v, sem_nt)
    bt_dma = pltpu.async_copy(bt_hbm.at[wid], bt_v, sem_bt)
    pltpu.sync_copy(tbl_hbm, tbl_v)

    @pl.when(sid == 0)
    def _():
        zeros = jnp.zeros((16,), jnp.float32)

        def zstep(i, carry):
            zero_v[pl.ds(i * 16, 16)] = zeros
            return carry

        lax.fori_loop(0, _G // 16, zstep, 0)
        pltpu.sync_copy(zero_v, acc_sh)

    plsc.subcore_barrier()
    nt_dma.wait()
    bt_dma.wait()

    def row_step(j, carry):
        for k in range(_VECS_PER_ROW):
            idx = nt_v[j, pl.ds(k * 16, 16)]
            val_v[j, pl.ds(k * 16, 16)] = plsc.load_gather(tbl_v, [idx])
        pltpu.async_copy(val_v.at[j], acc_sh.at[bt_v.at[j]], sem_sc,
                         add=True)
        return carry

    lax.fori_loop(0, _ROWS_PER_W, row_step, 0)

    def drain_step(j, carry):
        pltpu.make_async_copy(val_v.at[0], acc_sh.at[bt_v.at[0]],
                              sem_sc).wait()
        return carry

    lax.fori_loop(0, _ROWS_PER_W, drain_step, 0)

    plsc.subcore_barrier()

    @pl.when(sid == 0)
    def _():
        pltpu.sync_copy(acc_sh, out_hbm.at[cid])


@functools.cache
def _sc_call():
    mesh = plsc.VectorSubcoreMesh(
        core_axis_name="c", subcore_axis_name="s",
        num_cores=_NC, num_subcores=_NS)
    return pl.kernel(
        _sc_body,
        out_type=jax.ShapeDtypeStruct((_NC, _G), jnp.float32),
        mesh=mesh,
        compiler_params=pltpu.CompilerParams(needs_layout_passes=False, use_tc_tiling_on_sc=False),
        scratch_types=[
            pltpu.VMEM((_ROWS_PER_W, _CW), jnp.int32),
            pltpu.VMEM((_ROWS_PER_W, _CW), jnp.int32),
            pltpu.VMEM((_ROWS_PER_W, _CW), jnp.float32),
            pltpu.VMEM((_TBL,), jnp.float32),
            pltpu.VMEM((_G,), jnp.float32),
            pltpu.VMEM_SHARED((_G,), jnp.float32),
            pltpu.SemaphoreType.DMA,
            pltpu.SemaphoreType.DMA,
            pltpu.SemaphoreType.DMA,
        ],
    )


def kernel(node_type, batch, property_offset):
    nt = node_type.astype(jnp.int32)
    bt = batch.astype(jnp.int32)
    tbl = jnp.zeros((_TBL,), jnp.float32).at[:_Z].set(
        property_offset.astype(jnp.float32))
    pad = _NPAD - _N
    nt = jnp.concatenate([nt, jnp.full((pad,), _Z, jnp.int32)])
    bt = jnp.concatenate([bt, jnp.zeros((pad,), jnp.int32)])
    nt = nt.reshape(_NW, _ROWS_PER_W, _CW)
    bt = bt.reshape(_NW, _ROWS_PER_W, _CW)
    partial = _sc_call()(nt, bt, tbl)
    return partial[0] + partial[1]


# speedup vs baseline: 1.0848x; 1.0848x over previous
import functools

import jax
import jax.numpy as jnp
from jax import lax
from jax.experimental import pallas as pl
from jax.experimental.pallas import tpu as pltpu
from jax.experimental.pallas import tpu_sc as plsc

_N = 100000
_G = 1024
_Z = 89
_TBL = 128
_PADIDX = 96
_NC = 2
_NS = 16
_NW = _NC * _NS
_CW = 128
_ROWS_PER_W = 25
_CHUNK = _ROWS_PER_W * _CW
_LAST_BASE = (_NW - 1) * _CHUNK
_LAST_REAL = _N - _LAST_BASE
_LAST_ROWS = -(-_LAST_REAL // _CW)
_VECS_PER_ROW = _CW // 16


def _sc_body(nt_hbm, bt_hbm, tbl_hbm, out_hbm,
             nt_v, bt_v, val_v, bt2_v, tbl_v, zero_v, acc_sh,
             sem_nt, sem_bt, sem_tb, sem_sc):
    cid = lax.axis_index("c")
    sid = lax.axis_index("s")
    wid = sid * _NC + cid
    is_last = wid == _NW - 1
    base = wid * _CHUNK

    pltpu.async_copy(tbl_hbm, tbl_v.at[pl.ds(0, _Z)], sem_tb)

    @pl.when(jnp.logical_not(is_last))
    def _():
        pltpu.async_copy(nt_hbm.at[pl.ds(base, _CHUNK)], nt_v, sem_nt)
        pltpu.async_copy(bt_hbm.at[pl.ds(base, _CHUNK)], bt_v, sem_bt)

    @pl.when(is_last)
    def _():
        pltpu.async_copy(nt_hbm.at[pl.ds(_LAST_BASE, _LAST_REAL)],
                         nt_v.at[pl.ds(0, _LAST_REAL)], sem_nt)
        pltpu.async_copy(bt_hbm.at[pl.ds(_LAST_BASE, _LAST_REAL)],
                         bt_v.at[pl.ds(0, _LAST_REAL)], sem_bt)

    @pl.when(sid == 0)
    def _():
        zeros = jnp.zeros((16,), jnp.float32)

        def zstep(i, carry):
            zero_v[pl.ds(i * 16, 16)] = zeros
            return carry

        lax.fori_loop(0, _G // 16, zstep, 0)
        pltpu.sync_copy(zero_v, acc_sh)

    plsc.subcore_barrier()

    pltpu.make_async_copy(tbl_hbm, tbl_v.at[pl.ds(0, _Z)], sem_tb).wait()
    tbl_v[pl.ds(_PADIDX, 16)] = jnp.zeros((16,), jnp.float32)

    @pl.when(jnp.logical_not(is_last))
    def _():
        pltpu.make_async_copy(nt_hbm.at[pl.ds(0, _CHUNK)], nt_v,
                              sem_nt).wait()
        pltpu.make_async_copy(bt_hbm.at[pl.ds(0, _CHUNK)], bt_v,
                              sem_bt).wait()

    @pl.when(is_last)
    def _():
        pltpu.make_async_copy(nt_hbm.at[pl.ds(0, _LAST_REAL)],
                              nt_v.at[pl.ds(0, _LAST_REAL)], sem_nt).wait()
        pltpu.make_async_copy(bt_hbm.at[pl.ds(0, _LAST_REAL)],
                              bt_v.at[pl.ds(0, _LAST_REAL)], sem_bt).wait()
        for k in range(_LAST_REAL // 16, _LAST_ROWS * _VECS_PER_ROW):
            nt_v[pl.ds(k * 16, 16)] = jnp.full((16,), _PADIDX, jnp.int32)
            bt_v[pl.ds(k * 16, 16)] = jnp.zeros((16,), jnp.int32)

    nrows = jnp.where(is_last, _LAST_ROWS, _ROWS_PER_W)

    def row_step(j, carry):
        for k in range(_VECS_PER_ROW):
            off = j * _CW + k * 16
            idx = nt_v[pl.ds(off, 16)]
            val_v[pl.ds(off, 16)] = plsc.load_gather(tbl_v, [idx])
            bt2_v[j, pl.ds(k * 16, 16)] = bt_v[pl.ds(off, 16)]
        pltpu.async_copy(val_v.at[pl.ds(j * _CW, _CW)],
                         acc_sh.at[bt2_v.at[j]], sem_sc, add=True)
        return carry

    lax.fori_loop(0, nrows, row_step, 0)

    def drain_step(j, carry):
        pltpu.make_async_copy(val_v.at[pl.ds(0, _CW)],
                              acc_sh.at[bt2_v.at[0]], sem_sc).wait()
        return carry

    lax.fori_loop(0, nrows, drain_step, 0)

    plsc.subcore_barrier()

    @pl.when(sid == 0)
    def _():
        pltpu.sync_copy(acc_sh, out_hbm.at[cid])


@functools.cache
def _sc_call():
    mesh = plsc.VectorSubcoreMesh(
        core_axis_name="c", subcore_axis_name="s",
        num_cores=_NC, num_subcores=_NS)
    return pl.kernel(
        _sc_body,
        out_type=jax.ShapeDtypeStruct((_NC, _G), jnp.float32),
        mesh=mesh,
        compiler_params=pltpu.CompilerParams(needs_layout_passes=False),
        scratch_types=[
            pltpu.VMEM((_CHUNK,), jnp.int32),
            pltpu.VMEM((_CHUNK,), jnp.int32),
            pltpu.VMEM((_CHUNK,), jnp.float32),
            pltpu.VMEM((_ROWS_PER_W, _CW), jnp.int32),
            pltpu.VMEM((_TBL,), jnp.float32),
            pltpu.VMEM((_G,), jnp.float32),
            pltpu.VMEM_SHARED((_G,), jnp.float32),
            pltpu.SemaphoreType.DMA,
            pltpu.SemaphoreType.DMA,
            pltpu.SemaphoreType.DMA,
            pltpu.SemaphoreType.DMA,
        ],
    )


def kernel(node_type, batch, property_offset):
    nt = node_type.astype(jnp.int32)
    bt = batch.astype(jnp.int32)
    tbl = property_offset.astype(jnp.float32)
    partial = _sc_call()(nt, bt, tbl)
    return partial[0] + partial[1]
